# baseline (speedup 1.0000x reference)
import functools

import jax
import jax.numpy as jnp
from jax.experimental import pallas as pl
from jax.experimental.pallas import tpu as pltpu

V = 100000
VB = 2000
NBLK = V // VB


def _layer_norm(x, w, b, eps=1e-5):
    mu = jnp.mean(x, axis=-1, keepdims=True)
    var = jnp.mean((x - mu) ** 2, axis=-1, keepdims=True)
    return (x - mu) / jnp.sqrt(var + eps) * w + b


def _bilinear_sample(feat, u_norm, v_norm):
    C, Hf, Wf = feat.shape
    x = (u_norm + 1.0) * 0.5 * (Wf - 1)
    y = (v_norm + 1.0) * 0.5 * (Hf - 1)
    x0 = jnp.floor(x); y0 = jnp.floor(y)
    x1 = x0 + 1.0; y1 = y0 + 1.0
    wx1 = x - x0; wx0 = 1.0 - wx1
    wy1 = y - y0; wy0 = 1.0 - wy1
    def tap(xi, yi, wgt):
        valid = (xi >= 0) & (xi <= Wf - 1) & (yi >= 0) & (yi <= Hf - 1)
        xc = jnp.clip(xi, 0, Wf - 1).astype(jnp.int32)
        yc = jnp.clip(yi, 0, Hf - 1).astype(jnp.int32)
        return feat[:, yc, xc] * jnp.where(valid, wgt, 0.0)[None, :]
    out = tap(x0, y0, wx0 * wy0) + tap(x1, y0, wx1 * wy0) + tap(x0, y1, wx0 * wy1) + tap(x1, y1, wx1 * wy1)
    return out.T


def _project_feat(voxel_coords, calib_matrices, img_feats, Himg, Wimg):
    Vn = voxel_coords.shape[0]
    cams = img_feats.shape[0]; C = img_feats.shape[1]
    vis_acc = jnp.zeros((Vn, C), dtype=jnp.float32)
    vis_cnt = jnp.zeros((Vn, 1), dtype=jnp.float32)
    homo = jnp.concatenate([voxel_coords, jnp.ones((Vn, 1), dtype=voxel_coords.dtype)], axis=1)
    for cam in range(cams):
        RT = calib_matrices[cam]
        proj = homo @ RT.T
        u = proj[:, 0]; v = proj[:, 1]; w = proj[:, 2]
        u = u / jnp.clip(w, 1e-6, None)
        v = v / jnp.clip(w, 1e-6, None)
        u_norm = u / (Wimg / 2.0) - 1.0
        v_norm = v / (Himg / 2.0) - 1.0
        sampled = _bilinear_sample(img_feats[cam], u_norm, v_norm)
        mask = (w > 0) & (u >= 0) & (u < Wimg) & (v >= 0) & (v < Himg)
        vis_acc = vis_acc + jnp.where(mask[:, None], sampled, 0.0)
        vis_cnt = vis_cnt + mask[:, None].astype(jnp.float32)
    vis_cnt = jnp.where(vis_cnt == 0, 1.0, vis_cnt)
    return vis_acc / vis_cnt


def _bev_scatter(feats, coords):
    x_min, x_max, y_min, y_max, res = -50.0, 50.0, -50.0, 50.0, 0.5
    x = coords[:, 0]; y = coords[:, 1]
    Hbev = int((y_max - y_min) / res); Wbev = int((x_max - x_min) / res)
    ix = jnp.floor((x - x_min) / res).astype(jnp.int32)
    iy = jnp.floor((y - y_min) / res).astype(jnp.int32)
    mask = (ix >= 0) & (ix < Wbev) & (iy >= 0) & (iy < Hbev)
    idx_flat = iy * Wbev + ix
    idx_safe = jnp.where(mask, idx_flat, 0)
    wv = mask.astype(feats.dtype)
    C = feats.shape[1]
    bev = jnp.zeros((Hbev * Wbev, C), dtype=feats.dtype).at[idx_safe].add(feats * wv[:, None])
    cnt = jnp.zeros((Hbev * Wbev, 1), dtype=feats.dtype).at[idx_safe].add(wv[:, None])
    cnt = jnp.where(cnt == 0, 1.0, cnt)
    bev = (bev / cnt).reshape(Hbev, Wbev, C).transpose(2, 0, 1)
    return bev, idx_flat, mask, (Hbev, Wbev)


def _conv3x3(x, w, b):
    y = jax.lax.conv_general_dilated(x[None], w, (1, 1), [(1, 1), (1, 1)], dimension_numbers=('NCHW', 'OIHW', 'NCHW'))
    return y[0] + b[:, None, None]


def _final_body(x_ref, w_ref, b_ref, wsT_ref, bs_ref, o_ref):
    x = x_ref[...]
    mu = jnp.mean(x, axis=-1, keepdims=True)
    var = jnp.mean((x - mu) ** 2, axis=-1, keepdims=True)
    y = (x - mu) / jnp.sqrt(var + 1e-5) * w_ref[...] + b_ref[...]
    o_ref[...] = jnp.dot(y, wsT_ref[...], preferred_element_type=jnp.float32) + bs_ref[...]


def _final_stage(pts_feat, ln_o_w, ln_o_b, WsT, bs):
    return pl.pallas_call(
        _final_body,
        grid=(NBLK,),
        in_specs=[
            pl.BlockSpec((VB, 128), lambda i: (i, 0)),
            pl.BlockSpec((1, 128), lambda i: (0, 0)),
            pl.BlockSpec((1, 128), lambda i: (0, 0)),
            pl.BlockSpec((128, 32), lambda i: (0, 0)),
            pl.BlockSpec((1, 32), lambda i: (0, 0)),
        ],
        out_specs=pl.BlockSpec((VB, 32), lambda i: (i, 0)),
        out_shape=jax.ShapeDtypeStruct((V, 32), jnp.float32),
    )(pts_feat, ln_o_w.reshape(1, 128), ln_o_b.reshape(1, 128), WsT, bs.reshape(1, 32))


def kernel(point_features, image_features, voxel_coords, calib_matrices, img_shape, feat_shape, ln_l_w, ln_l_b, W_l, b_l, ln_i_w, ln_i_b, W_i, b_i, Wg1, bg1, Wg2, bg2, Wc1, bc1, Wc2, bc2, alpha, ln_o_w, ln_o_b, Ws, bs):
    Himg = img_shape[0].astype(jnp.float32); Wimg = img_shape[1].astype(jnp.float32)
    vis_feat = _project_feat(voxel_coords, calib_matrices, image_features, Himg, Wimg)
    lidar_emb = jax.nn.relu(_layer_norm(point_features, ln_l_w, ln_l_b) @ W_l.T + b_l)
    img_emb = jax.nn.relu(_layer_norm(vis_feat, ln_i_w, ln_i_b) @ W_i.T + b_i)
    g_h = jax.nn.relu(jnp.concatenate([lidar_emb, img_emb], axis=-1) @ Wg1.T + bg1)
    gate = jax.nn.sigmoid(g_h @ Wg2.T + bg2)
    fused_pt = gate * img_emb + (1.0 - gate) * lidar_emb
    bev_lidar, idx_flat, mask_inside, (Hbev, Wbev) = _bev_scatter(fused_pt, voxel_coords)
    bev_img, _, _, _ = _bev_scatter(img_emb, voxel_coords)
    h = jax.nn.relu(_conv3x3(bev_img, Wc1, bc1))
    bev_fused = bev_lidar + 0.5 * _conv3x3(h, Wc2, bc2)
    bev_flat = bev_fused.transpose(1, 2, 0).reshape(-1, bev_fused.shape[0])
    gathered = bev_flat[jnp.clip(idx_flat, 0, Hbev * Wbev - 1)]
    pts_feat = jnp.where(mask_inside[:, None], fused_pt + jnp.clip(alpha, 0.0, 1.0) * gathered, fused_pt)
    return _final_stage(pts_feat, ln_o_w, ln_o_b, Ws.T, bs)


# speedup vs baseline: 3.5548x; 3.5548x over previous
import functools

import jax
import jax.numpy as jnp
from jax import lax
from jax.experimental import pallas as pl
from jax.experimental.pallas import tpu as pltpu

V = 100000
VB = 1024
NBLK = 100
V_PAD = VB * NBLK
KPAD = 1408
HW = 37
NCAM = 6
C_IMG = 384
C_EMB = 128
CELLS = 40000
RPAD = 1024
TROWS = 43008
DUMP = RPAD + 40960
NTB = 40960


def _stage_a(vox_p, pf_p, feat_pad, calib_cat, scal, lnlw, lnlb, WlT, bl,
             lniw, lnib, WiT, bi, Wg1aT, Wg1bT, bg1, wg2row, bg2):
    def body(vox_ref, pf_ref, feat_ref, cc_ref, sc_ref,
             lnlw_ref, lnlb_ref, wlT_ref, bl_ref,
             lniw_ref, lnib_ref, wiT_ref, bi_ref,
             wg1a_ref, wg1b_ref, bg1_ref, wg2_ref, bg2_ref,
             fused_ref, img_ref, idxs_ref, idxg_ref):
        i = pl.program_id(0)
        vox = vox_ref[...]

        def tb(t):
            return t.astype(jnp.bfloat16).astype(jnp.float32)

        vx = tb(vox[:, 0:1]); vy = tb(vox[:, 1:2]); vz = tb(vox[:, 2:3])
        su = sc_ref[0:1, 0:1]; sv = sc_ref[0:1, 1:2]
        wimg = sc_ref[0:1, 2:3]; himg = sc_ref[0:1, 3:4]

        iota_k = lax.broadcasted_iota(jnp.int32, (VB, KPAD), 1)
        acc = jnp.zeros((VB, C_IMG), dtype=jnp.float32)
        cnt = jnp.zeros((VB, 1), dtype=jnp.float32)
        for cam in range(NCAM):
            u = (vx * tb(cc_ref[0:1, 3 * cam:3 * cam + 1])
                 + vy * tb(cc_ref[1:2, 3 * cam:3 * cam + 1])
                 + vz * tb(cc_ref[2:3, 3 * cam:3 * cam + 1])
                 + tb(cc_ref[3:4, 3 * cam:3 * cam + 1]))
            v = (vx * tb(cc_ref[0:1, 3 * cam + 1:3 * cam + 2])
                 + vy * tb(cc_ref[1:2, 3 * cam + 1:3 * cam + 2])
                 + vz * tb(cc_ref[2:3, 3 * cam + 1:3 * cam + 2])
                 + tb(cc_ref[3:4, 3 * cam + 1:3 * cam + 2]))
            w = (vx * tb(cc_ref[0:1, 3 * cam + 2:3 * cam + 3])
                 + vy * tb(cc_ref[1:2, 3 * cam + 2:3 * cam + 3])
                 + vz * tb(cc_ref[2:3, 3 * cam + 2:3 * cam + 3])
                 + tb(cc_ref[3:4, 3 * cam + 2:3 * cam + 3]))
            wc = jnp.maximum(w, 1e-6)
            ud = u / wc
            vd = v / wc
            un = ud * su - 1.0
            vn = vd * sv - 1.0
            x = (un + 1.0) * 0.5 * (HW - 1.0)
            y = (vn + 1.0) * 0.5 * (HW - 1.0)
            x0 = jnp.floor(x); y0 = jnp.floor(y)
            wx1 = x - x0; wx0 = 1.0 - wx1
            wy1 = y - y0; wy0 = 1.0 - wy1
            vx0 = ((x0 >= 0.0) & (x0 <= HW - 1.0)).astype(jnp.float32)
            vx1 = ((x0 + 1.0 >= 0.0) & (x0 + 1.0 <= HW - 1.0)).astype(jnp.float32)
            vy0 = ((y0 >= 0.0) & (y0 <= HW - 1.0)).astype(jnp.float32)
            vy1 = ((y0 + 1.0 >= 0.0) & (y0 + 1.0 <= HW - 1.0)).astype(jnp.float32)
            x0i = jnp.clip(x0, -2.0, 38.0).astype(jnp.int32)
            y0i = jnp.clip(y0, -2.0, 38.0).astype(jnp.int32)
            f00 = y0i * HW + x0i
            w00 = wx0 * wy0 * vx0 * vy0
            w10 = wx1 * wy0 * vx1 * vy0
            w01 = wx0 * wy1 * vx0 * vy1
            w11 = wx1 * wy1 * vx1 * vy1
            s = jnp.where(iota_k == f00, w00, 0.0)
            s = s + jnp.where(iota_k == f00 + 1, w10, 0.0)
            s = s + jnp.where(iota_k == f00 + HW, w01, 0.0)
            s = s + jnp.where(iota_k == f00 + HW + 1, w11, 0.0)
            sampled = jnp.dot(s, feat_ref[cam],
                              preferred_element_type=jnp.float32, precision=lax.Precision.HIGHEST)
            m = ((w > 0.0) & (ud >= 0.0) & (ud < wimg)
                 & (vd >= 0.0) & (vd < himg)).astype(jnp.float32)
            acc = acc + m * sampled
            cnt = cnt + m
        cntc = jnp.where(cnt == 0.0, 1.0, cnt)
        vis = acc / cntc

        mu = jnp.mean(vis, axis=-1, keepdims=True)
        var = jnp.mean((vis - mu) ** 2, axis=-1, keepdims=True)
        lnv = (vis - mu) / jnp.sqrt(var + 1e-5) * lniw_ref[...] + lnib_ref[...]
        img = jax.nn.relu(
            jnp.dot(lnv, wiT_ref[...], preferred_element_type=jnp.float32, precision=lax.Precision.HIGHEST)
            + bi_ref[...])

        pf = pf_ref[...]
        mul = jnp.mean(pf, axis=-1, keepdims=True)
        varl = jnp.mean((pf - mul) ** 2, axis=-1, keepdims=True)
        lnp = (pf - mul) / jnp.sqrt(varl + 1e-5) * lnlw_ref[...] + lnlb_ref[...]
        lid = jax.nn.relu(
            jnp.dot(lnp, wlT_ref[...], preferred_element_type=jnp.float32, precision=lax.Precision.HIGHEST)
            + bl_ref[...])

        g_h = jax.nn.relu(
            jnp.dot(lid, wg1a_ref[...], preferred_element_type=jnp.float32, precision=lax.Precision.HIGHEST)
            + jnp.dot(img, wg1b_ref[...], preferred_element_type=jnp.float32, precision=lax.Precision.HIGHEST)
            + bg1_ref[...])
        gate = jax.nn.sigmoid(
            jnp.sum(g_h * wg2_ref[...], axis=-1, keepdims=True) + bg2_ref[...])
        fused = gate * img + (1.0 - gate) * lid

        bx = vox[:, 0:1]
        by = vox[:, 1:2]
        ix = jnp.floor((bx + 50.0) * 2.0).astype(jnp.int32)
        iy = jnp.floor((by + 50.0) * 2.0).astype(jnp.int32)
        inb = (ix >= 0) & (ix < 200) & (iy >= 0) & (iy < 200)
        rowid = i * VB + lax.broadcasted_iota(jnp.int32, (VB, 1), 0)
        realrow = rowid < V
        idxf = iy * 200 + ix + RPAD
        idxs_ref[...] = jnp.where(inb & realrow, idxf, DUMP)
        idxg_ref[...] = jnp.where(inb & realrow, idxf - RPAD, 0)
        fused_ref[...] = fused
        img_ref[...] = img

    row1 = lambda i: (0, 0)
    return pl.pallas_call(
        body,
        grid=(NBLK,),
        in_specs=[
            pl.BlockSpec((VB, 3), lambda i: (i, 0)),
            pl.BlockSpec((VB, 16), lambda i: (i, 0)),
            pl.BlockSpec((NCAM, KPAD, C_IMG), lambda i: (0, 0, 0)),
            pl.BlockSpec((4, 18), row1),
            pl.BlockSpec((1, 4), row1),
            pl.BlockSpec((1, 16), row1),
            pl.BlockSpec((1, 16), row1),
            pl.BlockSpec((16, C_EMB), row1),
            pl.BlockSpec((1, C_EMB), row1),
            pl.BlockSpec((1, C_IMG), row1),
            pl.BlockSpec((1, C_IMG), row1),
            pl.BlockSpec((C_IMG, C_EMB), row1),
            pl.BlockSpec((1, C_EMB), row1),
            pl.BlockSpec((C_EMB, C_EMB), row1),
            pl.BlockSpec((C_EMB, C_EMB), row1),
            pl.BlockSpec((1, C_EMB), row1),
            pl.BlockSpec((1, C_EMB), row1),
            pl.BlockSpec((1, 1), row1),
        ],
        out_specs=[
            pl.BlockSpec((VB, C_EMB), lambda i: (i, 0)),
            pl.BlockSpec((VB, C_EMB), lambda i: (i, 0)),
            pl.BlockSpec((VB, 1), lambda i: (i, 0)),
            pl.BlockSpec((VB, 1), lambda i: (i, 0)),
        ],
        out_shape=[
            jax.ShapeDtypeStruct((V_PAD, C_EMB), jnp.float32),
            jax.ShapeDtypeStruct((V_PAD, C_EMB), jnp.float32),
            jax.ShapeDtypeStruct((V_PAD, 1), jnp.int32),
            jax.ShapeDtypeStruct((V_PAD, 1), jnp.int32),
        ],
    )(vox_p, pf_p, feat_pad, calib_cat, scal, lnlw, lnlb, WlT, bl,
      lniw, lnib, WiT, bi, Wg1aT, Wg1bT, bg1, wg2row, bg2)


def _c1a_body(simg_ref, cnt_ref, out_ref):
    i = pl.program_id(0)
    p = i * 1024 + lax.broadcasted_iota(jnp.int32, (1024, 1), 0)
    valid = (p >= RPAD) & (p < RPAD + CELLS)
    c = cnt_ref[:, 0:1]
    cc = jnp.where(c == 0.0, 1.0, c)
    out_ref[...] = jnp.where(valid, simg_ref[...] / cc, 0.0)


def _c1a(sum_img, cnt_tbl):
    return pl.pallas_call(
        _c1a_body,
        grid=(TROWS // 1024,),
        in_specs=[
            pl.BlockSpec((1024, C_EMB), lambda i: (i, 0)),
            pl.BlockSpec((1024, 16), lambda i: (i, 0)),
        ],
        out_specs=pl.BlockSpec((1024, C_EMB), lambda i: (i, 0)),
        out_shape=jax.ShapeDtypeStruct((TROWS, C_EMB), jnp.float32),
    )(sum_img, cnt_tbl)


def _c1b_body(sf_ref, cnt_ref, out_ref):
    c = cnt_ref[:, 0:1]
    cc = jnp.where(c == 0.0, 1.0, c)
    out_ref[...] = sf_ref[...] / cc


def _c1b(sum_fused, cnt_tbl):
    return pl.pallas_call(
        _c1b_body,
        grid=(40960 // 1024,),
        in_specs=[
            pl.BlockSpec((1024, C_EMB), lambda i: (i + 1, 0)),
            pl.BlockSpec((1024, 16), lambda i: (i + 1, 0)),
        ],
        out_specs=pl.BlockSpec((1024, C_EMB), lambda i: (i, 0)),
        out_shape=jax.ShapeDtypeStruct((40960, C_EMB), jnp.float32),
    )(sum_fused, cnt_tbl)


RB = 4000
HB = 4408
SHIFTS = [(dy, dx) for dy in (-1, 0, 1) for dx in (-1, 0, 1)]


def _c2_body(ximg_ref, xf_ref, w1_ref, b1_ref, w2_ref, b2_ref, out_ref,
             h_ref):
    i = pl.program_id(0)
    r0 = i * RB
    jj = lax.broadcasted_iota(jnp.int32, (HB, 1), 0)
    q2 = (r0 - 204) + jj + 1600
    wq = lax.rem(q2, 200)
    acc = jnp.zeros((HB, C_EMB), dtype=jnp.float32)
    for k, (dy, dx) in enumerate(SHIFTS):
        sft = dy * 200 + dx
        start = RPAD + (r0 - 204) + sft
        xs = ximg_ref[pl.ds(start, HB), :]
        t = jnp.dot(xs, w1_ref[k], preferred_element_type=jnp.float32, precision=lax.Precision.HIGHEST)
        if dx == 1:
            t = jnp.where(wq == 199, 0.0, t)
        elif dx == -1:
            t = jnp.where(wq == 0, 0.0, t)
        acc = acc + t
    h_ref[...] = jax.nn.relu(acc + b1_ref[...])

    kk = lax.broadcasted_iota(jnp.int32, (RB, 1), 0)
    p = r0 + kk
    wp = lax.rem(p, 200)
    acc2 = jnp.zeros((RB, C_EMB), dtype=jnp.float32)
    for k, (dy, dx) in enumerate(SHIFTS):
        sft = dy * 200 + dx
        hs = h_ref[pl.ds(204 + sft, RB), :]
        t = jnp.dot(hs, w2_ref[k], preferred_element_type=jnp.float32, precision=lax.Precision.HIGHEST)
        okr = None
        if dy == -1:
            okr = p >= 200
        elif dy == 1:
            okr = p < CELLS - 200
        okc = None
        if dx == 1:
            okc = wp != 199
        elif dx == -1:
            okc = wp != 0
        if okr is not None and okc is not None:
            t = jnp.where(okr & okc, t, 0.0)
        elif okr is not None:
            t = jnp.where(okr, t, 0.0)
        elif okc is not None:
            t = jnp.where(okc, t, 0.0)
        acc2 = acc2 + t
    out_ref[...] = xf_ref[...] + 0.5 * (acc2 + b2_ref[...])


def _c2(ximg_pad, xfused, W1m, b1, W2m, b2):
    return pl.pallas_call(
        _c2_body,
        grid=(CELLS // RB,),
        in_specs=[
            pl.BlockSpec((TROWS, C_EMB), lambda i: (0, 0)),
            pl.BlockSpec((RB, C_EMB), lambda i: (i, 0)),
            pl.BlockSpec((9, C_EMB, C_EMB), lambda i: (0, 0, 0)),
            pl.BlockSpec((1, C_EMB), lambda i: (0, 0)),
            pl.BlockSpec((9, C_EMB, C_EMB), lambda i: (0, 0, 0)),
            pl.BlockSpec((1, C_EMB), lambda i: (0, 0)),
        ],
        out_specs=pl.BlockSpec((RB, C_EMB), lambda i: (i, 0)),
        out_shape=jax.ShapeDtypeStruct((CELLS, C_EMB), jnp.float32),
        scratch_shapes=[pltpu.VMEM((HB, C_EMB), jnp.float32)],
    )(ximg_pad, xfused, W1m, b1, W2m, b2)


def _e_body(f_ref, g_ref, idxs_ref, al_ref, w_ref, b_ref, wsT_ref, bs_ref,
            o_ref):
    mask = idxs_ref[...] != DUMP
    x = jnp.where(mask, f_ref[...] + al_ref[...] * g_ref[...], f_ref[...])
    mu = jnp.mean(x, axis=-1, keepdims=True)
    var = jnp.mean((x - mu) ** 2, axis=-1, keepdims=True)
    y = (x - mu) / jnp.sqrt(var + 1e-5) * w_ref[...] + b_ref[...]
    o_ref[...] = jnp.dot(y, wsT_ref[...], preferred_element_type=jnp.float32, precision=lax.Precision.HIGHEST) \
        + bs_ref[...]


def _stage_e(fused, gathered, idx_s, alphac, ln_o_w, ln_o_b, WsT, bs):
    row1 = lambda i: (0, 0)
    return pl.pallas_call(
        _e_body,
        grid=(NBLK,),
        in_specs=[
            pl.BlockSpec((VB, C_EMB), lambda i: (i, 0)),
            pl.BlockSpec((VB, C_EMB), lambda i: (i, 0)),
            pl.BlockSpec((VB, 1), lambda i: (i, 0)),
            pl.BlockSpec((1, 1), row1),
            pl.BlockSpec((1, C_EMB), row1),
            pl.BlockSpec((1, C_EMB), row1),
            pl.BlockSpec((C_EMB, 32), row1),
            pl.BlockSpec((1, 32), row1),
        ],
        out_specs=pl.BlockSpec((VB, 32), lambda i: (i, 0)),
        out_shape=jax.ShapeDtypeStruct((V_PAD, 32), jnp.float32),
    )(fused, gathered, idx_s, alphac, ln_o_w.reshape(1, C_EMB),
      ln_o_b.reshape(1, C_EMB), WsT, bs.reshape(1, 32))


def kernel(point_features, image_features, voxel_coords, calib_matrices,
           img_shape, feat_shape, ln_l_w, ln_l_b, W_l, b_l, ln_i_w, ln_i_b,
           W_i, b_i, Wg1, bg1, Wg2, bg2, Wc1, bc1, Wc2, bc2, alpha,
           ln_o_w, ln_o_b, Ws, bs):
    f32 = jnp.float32
    feat_hw = image_features.transpose(0, 2, 3, 1).reshape(NCAM, HW * HW,
                                                           C_IMG)
    feat_pad = jnp.pad(feat_hw, ((0, 0), (0, KPAD - HW * HW), (0, 0)))
    calib_cat = jnp.transpose(calib_matrices, (2, 0, 1)).reshape(4, 18)
    Wimg = img_shape[1].astype(f32)
    Himg = img_shape[0].astype(f32)
    scal = jnp.stack([2.0 / Wimg, 2.0 / Himg, Wimg, Himg]).reshape(1, 4)
    vox_p = jnp.pad(voxel_coords, ((0, V_PAD - V), (0, 0)))
    pf_p = jnp.pad(point_features, ((0, V_PAD - V), (0, 0)))
    Wg1T = Wg1.T
    fused, img_emb, idx_s, idx_g = _stage_a(
        vox_p, pf_p, feat_pad, calib_cat, scal,
        ln_l_w.reshape(1, 16), ln_l_b.reshape(1, 16), W_l.T,
        b_l.reshape(1, C_EMB),
        ln_i_w.reshape(1, C_IMG), ln_i_b.reshape(1, C_IMG), W_i.T,
        b_i.reshape(1, C_EMB),
        Wg1T[:C_EMB], Wg1T[C_EMB:], bg1.reshape(1, C_EMB),
        Wg2.reshape(1, C_EMB), bg2.reshape(1, 1))

    idx_flat = idx_s.reshape(V_PAD)
    sum_fused = jnp.zeros((TROWS, C_EMB), f32).at[idx_flat].add(fused)
    sum_img = jnp.zeros((TROWS, C_EMB), f32).at[idx_flat].add(img_emb)
    cnt_tbl = jnp.zeros((TROWS, 16), f32).at[idx_flat].add(
        jnp.ones((V_PAD, 16), f32))

    ximg_pad = _c1a(sum_img, cnt_tbl)
    xfused = _c1b(sum_fused, cnt_tbl)
    W1m = jnp.stack([Wc1[:, :, dy + 1, dx + 1].T for dy, dx in SHIFTS])
    W2m = jnp.stack([Wc2[:, :, dy + 1, dx + 1].T for dy, dx in SHIFTS])
    bev_fused = _c2(ximg_pad, xfused, W1m, bc1.reshape(1, C_EMB), W2m,
                    bc2.reshape(1, C_EMB))

    gathered = bev_fused[idx_g.reshape(V_PAD)]

    alphac = jnp.clip(alpha, 0.0, 1.0).reshape(1, 1)
    logits = _stage_e(fused, gathered, idx_s, alphac, ln_o_w, ln_o_b,
                      Ws.T, bs)
    return logits[:V]


# speedup vs baseline: 7.2356x; 2.0355x over previous
import functools

import jax
import jax.numpy as jnp
from jax import lax
from jax.experimental import pallas as pl
from jax.experimental.pallas import tpu as pltpu

V = 100000
VB = 1024
NBLK = 100
V_PAD = VB * NBLK
KPAD = 1408
HW = 37
NCAM = 6
C_IMG = 384
C_EMB = 128
CELLS = 40000
RPAD = 1024
TROWS = 43008
DUMP = RPAD + 40960
NTB = 40960


def _stage_a(vox_p, pf_p, feat_pad, calib_cat, scal, lnlw, lnlb, WlT, bl,
             lniw, lnib, WiT, bi, Wg1aT, Wg1bT, bg1, wg2row, bg2):
    def body(vox_ref, pf_ref, feat_ref, cc_ref, sc_ref,
             lnlw_ref, lnlb_ref, wlT_ref, bl_ref,
             lniw_ref, lnib_ref, wiT_ref, bi_ref,
             wg1a_ref, wg1b_ref, bg1_ref, wg2_ref, bg2_ref,
             fused_ref, img_ref, idxs_ref, idxg_ref):
        i = pl.program_id(0)
        vox = vox_ref[...]

        def tb(t):
            return t.astype(jnp.bfloat16).astype(jnp.float32)

        vx = tb(vox[:, 0:1]); vy = tb(vox[:, 1:2]); vz = tb(vox[:, 2:3])
        su = sc_ref[0:1, 0:1]; sv = sc_ref[0:1, 1:2]
        wimg = sc_ref[0:1, 2:3]; himg = sc_ref[0:1, 3:4]

        iota_k = lax.broadcasted_iota(jnp.int32, (VB, KPAD), 1)
        acc = jnp.zeros((VB, C_IMG), dtype=jnp.float32)
        cnt = jnp.zeros((VB, 1), dtype=jnp.float32)
        for cam in range(NCAM):
            u = (vx * tb(cc_ref[0:1, 3 * cam:3 * cam + 1])
                 + vy * tb(cc_ref[1:2, 3 * cam:3 * cam + 1])
                 + vz * tb(cc_ref[2:3, 3 * cam:3 * cam + 1])
                 + tb(cc_ref[3:4, 3 * cam:3 * cam + 1]))
            v = (vx * tb(cc_ref[0:1, 3 * cam + 1:3 * cam + 2])
                 + vy * tb(cc_ref[1:2, 3 * cam + 1:3 * cam + 2])
                 + vz * tb(cc_ref[2:3, 3 * cam + 1:3 * cam + 2])
                 + tb(cc_ref[3:4, 3 * cam + 1:3 * cam + 2]))
            w = (vx * tb(cc_ref[0:1, 3 * cam + 2:3 * cam + 3])
                 + vy * tb(cc_ref[1:2, 3 * cam + 2:3 * cam + 3])
                 + vz * tb(cc_ref[2:3, 3 * cam + 2:3 * cam + 3])
                 + tb(cc_ref[3:4, 3 * cam + 2:3 * cam + 3]))
            wc = jnp.maximum(w, 1e-6)
            ud = u / wc
            vd = v / wc
            un = ud * su - 1.0
            vn = vd * sv - 1.0
            x = (un + 1.0) * 0.5 * (HW - 1.0)
            y = (vn + 1.0) * 0.5 * (HW - 1.0)
            x0 = jnp.floor(x); y0 = jnp.floor(y)
            wx1 = x - x0; wx0 = 1.0 - wx1
            wy1 = y - y0; wy0 = 1.0 - wy1
            vx0 = ((x0 >= 0.0) & (x0 <= HW - 1.0)).astype(jnp.float32)
            vx1 = ((x0 + 1.0 >= 0.0) & (x0 + 1.0 <= HW - 1.0)).astype(jnp.float32)
            vy0 = ((y0 >= 0.0) & (y0 <= HW - 1.0)).astype(jnp.float32)
            vy1 = ((y0 + 1.0 >= 0.0) & (y0 + 1.0 <= HW - 1.0)).astype(jnp.float32)
            x0i = jnp.clip(x0, -2.0, 38.0).astype(jnp.int32)
            y0i = jnp.clip(y0, -2.0, 38.0).astype(jnp.int32)
            f00 = y0i * HW + x0i
            w00 = wx0 * wy0 * vx0 * vy0
            w10 = wx1 * wy0 * vx1 * vy0
            w01 = wx0 * wy1 * vx0 * vy1
            w11 = wx1 * wy1 * vx1 * vy1
            s = jnp.where(iota_k == f00, w00, 0.0)
            s = s + jnp.where(iota_k == f00 + 1, w10, 0.0)
            s = s + jnp.where(iota_k == f00 + HW, w01, 0.0)
            s = s + jnp.where(iota_k == f00 + HW + 1, w11, 0.0)
            sampled = jnp.dot(s, feat_ref[cam],
                              preferred_element_type=jnp.float32)
            m = ((w > 0.0) & (ud >= 0.0) & (ud < wimg)
                 & (vd >= 0.0) & (vd < himg)).astype(jnp.float32)
            acc = acc + m * sampled
            cnt = cnt + m
        cntc = jnp.where(cnt == 0.0, 1.0, cnt)
        vis = acc / cntc

        mu = jnp.mean(vis, axis=-1, keepdims=True)
        var = jnp.mean((vis - mu) ** 2, axis=-1, keepdims=True)
        lnv = (vis - mu) / jnp.sqrt(var + 1e-5) * lniw_ref[...] + lnib_ref[...]
        img = jax.nn.relu(
            jnp.dot(lnv, wiT_ref[...], preferred_element_type=jnp.float32, precision=lax.Precision.HIGHEST)
            + bi_ref[...])

        pf = pf_ref[...]
        mul = jnp.mean(pf, axis=-1, keepdims=True)
        varl = jnp.mean((pf - mul) ** 2, axis=-1, keepdims=True)
        lnp = (pf - mul) / jnp.sqrt(varl + 1e-5) * lnlw_ref[...] + lnlb_ref[...]
        lid = jax.nn.relu(
            jnp.dot(lnp, wlT_ref[...], preferred_element_type=jnp.float32, precision=lax.Precision.HIGHEST)
            + bl_ref[...])

        g_h = jax.nn.relu(
            jnp.dot(lid, wg1a_ref[...], preferred_element_type=jnp.float32, precision=lax.Precision.HIGHEST)
            + jnp.dot(img, wg1b_ref[...], preferred_element_type=jnp.float32, precision=lax.Precision.HIGHEST)
            + bg1_ref[...])
        gate = jax.nn.sigmoid(
            jnp.sum(g_h * wg2_ref[...], axis=-1, keepdims=True) + bg2_ref[...])
        fused = gate * img + (1.0 - gate) * lid

        bx = vox[:, 0:1]
        by = vox[:, 1:2]
        ix = jnp.floor((bx + 50.0) * 2.0).astype(jnp.int32)
        iy = jnp.floor((by + 50.0) * 2.0).astype(jnp.int32)
        inb = (ix >= 0) & (ix < 200) & (iy >= 0) & (iy < 200)
        rowid = i * VB + lax.broadcasted_iota(jnp.int32, (VB, 1), 0)
        realrow = rowid < V
        idxf = iy * 200 + ix + RPAD
        idxs_ref[...] = jnp.where(inb & realrow, idxf, DUMP)
        idxg_ref[...] = jnp.where(inb & realrow, idxf - RPAD, 0)
        fused_ref[...] = fused
        img_ref[...] = img

    row1 = lambda i: (0, 0)
    return pl.pallas_call(
        body,
        grid=(NBLK,),
        in_specs=[
            pl.BlockSpec((VB, 3), lambda i: (i, 0)),
            pl.BlockSpec((VB, 16), lambda i: (i, 0)),
            pl.BlockSpec((NCAM, KPAD, C_IMG), lambda i: (0, 0, 0)),
            pl.BlockSpec((4, 18), row1),
            pl.BlockSpec((1, 4), row1),
            pl.BlockSpec((1, 16), row1),
            pl.BlockSpec((1, 16), row1),
            pl.BlockSpec((16, C_EMB), row1),
            pl.BlockSpec((1, C_EMB), row1),
            pl.BlockSpec((1, C_IMG), row1),
            pl.BlockSpec((1, C_IMG), row1),
            pl.BlockSpec((C_IMG, C_EMB), row1),
            pl.BlockSpec((1, C_EMB), row1),
            pl.BlockSpec((C_EMB, C_EMB), row1),
            pl.BlockSpec((C_EMB, C_EMB), row1),
            pl.BlockSpec((1, C_EMB), row1),
            pl.BlockSpec((1, C_EMB), row1),
            pl.BlockSpec((1, 1), row1),
        ],
        out_specs=[
            pl.BlockSpec((VB, C_EMB), lambda i: (i, 0)),
            pl.BlockSpec((VB, C_EMB), lambda i: (i, 0)),
            pl.BlockSpec((VB, 1), lambda i: (i, 0)),
            pl.BlockSpec((VB, 1), lambda i: (i, 0)),
        ],
        out_shape=[
            jax.ShapeDtypeStruct((V_PAD, C_EMB), jnp.float32),
            jax.ShapeDtypeStruct((V_PAD, C_EMB), jnp.float32),
            jax.ShapeDtypeStruct((V_PAD, 1), jnp.int32),
            jax.ShapeDtypeStruct((V_PAD, 1), jnp.int32),
        ],
    )(vox_p, pf_p, feat_pad, calib_cat, scal, lnlw, lnlb, WlT, bl,
      lniw, lnib, WiT, bi, Wg1aT, Wg1bT, bg1, wg2row, bg2)


def _c1a_body(simg_ref, cnt_ref, out_ref):
    i = pl.program_id(0)
    p = i * 1024 + lax.broadcasted_iota(jnp.int32, (1024, 1), 0)
    valid = (p >= RPAD) & (p < RPAD + CELLS)
    c = cnt_ref[:, 0:1]
    cc = jnp.where(c == 0.0, 1.0, c)
    out_ref[...] = jnp.where(valid, simg_ref[...] / cc, 0.0)


def _c1a(sum_img, cnt_tbl):
    return pl.pallas_call(
        _c1a_body,
        grid=(TROWS // 1024,),
        in_specs=[
            pl.BlockSpec((1024, C_EMB), lambda i: (i, 0)),
            pl.BlockSpec((1024, 16), lambda i: (i, 0)),
        ],
        out_specs=pl.BlockSpec((1024, C_EMB), lambda i: (i, 0)),
        out_shape=jax.ShapeDtypeStruct((TROWS, C_EMB), jnp.float32),
    )(sum_img, cnt_tbl)


def _c1b_body(sf_ref, cnt_ref, out_ref):
    c = cnt_ref[:, 0:1]
    cc = jnp.where(c == 0.0, 1.0, c)
    out_ref[...] = sf_ref[...] / cc


def _c1b(sum_fused, cnt_tbl):
    return pl.pallas_call(
        _c1b_body,
        grid=(40960 // 1024,),
        in_specs=[
            pl.BlockSpec((1024, C_EMB), lambda i: (i + 1, 0)),
            pl.BlockSpec((1024, 16), lambda i: (i + 1, 0)),
        ],
        out_specs=pl.BlockSpec((1024, C_EMB), lambda i: (i, 0)),
        out_shape=jax.ShapeDtypeStruct((40960, C_EMB), jnp.float32),
    )(sum_fused, cnt_tbl)


RB = 4000
HB = 4408
SHIFTS = [(dy, dx) for dy in (-1, 0, 1) for dx in (-1, 0, 1)]


def _c2_body(ximg_ref, xf_ref, w1_ref, b1_ref, w2_ref, b2_ref, out_ref,
             h_ref):
    i = pl.program_id(0)
    r0 = i * RB
    jj = lax.broadcasted_iota(jnp.int32, (HB, 1), 0)
    q2 = (r0 - 204) + jj + 1600
    wq = lax.rem(q2, 200)
    acc = jnp.zeros((HB, C_EMB), dtype=jnp.float32)
    for k, (dy, dx) in enumerate(SHIFTS):
        sft = dy * 200 + dx
        start = RPAD + (r0 - 204) + sft
        xs = ximg_ref[pl.ds(start, HB), :]
        t = jnp.dot(xs, w1_ref[k], preferred_element_type=jnp.float32)
        if dx == 1:
            t = jnp.where(wq == 199, 0.0, t)
        elif dx == -1:
            t = jnp.where(wq == 0, 0.0, t)
        acc = acc + t
    h_ref[...] = jax.nn.relu(acc + b1_ref[...])

    kk = lax.broadcasted_iota(jnp.int32, (RB, 1), 0)
    p = r0 + kk
    wp = lax.rem(p, 200)
    acc2 = jnp.zeros((RB, C_EMB), dtype=jnp.float32)
    for k, (dy, dx) in enumerate(SHIFTS):
        sft = dy * 200 + dx
        hs = h_ref[pl.ds(204 + sft, RB), :]
        t = jnp.dot(hs, w2_ref[k], preferred_element_type=jnp.float32)
        okr = None
        if dy == -1:
            okr = p >= 200
        elif dy == 1:
            okr = p < CELLS - 200
        okc = None
        if dx == 1:
            okc = wp != 199
        elif dx == -1:
            okc = wp != 0
        if okr is not None and okc is not None:
            t = jnp.where(okr & okc, t, 0.0)
        elif okr is not None:
            t = jnp.where(okr, t, 0.0)
        elif okc is not None:
            t = jnp.where(okc, t, 0.0)
        acc2 = acc2 + t
    out_ref[...] = xf_ref[...] + 0.5 * (acc2 + b2_ref[...])


def _c2(ximg_pad, xfused, W1m, b1, W2m, b2):
    return pl.pallas_call(
        _c2_body,
        grid=(CELLS // RB,),
        in_specs=[
            pl.BlockSpec((TROWS, C_EMB), lambda i: (0, 0)),
            pl.BlockSpec((RB, C_EMB), lambda i: (i, 0)),
            pl.BlockSpec((9, C_EMB, C_EMB), lambda i: (0, 0, 0)),
            pl.BlockSpec((1, C_EMB), lambda i: (0, 0)),
            pl.BlockSpec((9, C_EMB, C_EMB), lambda i: (0, 0, 0)),
            pl.BlockSpec((1, C_EMB), lambda i: (0, 0)),
        ],
        out_specs=pl.BlockSpec((RB, C_EMB), lambda i: (i, 0)),
        out_shape=jax.ShapeDtypeStruct((CELLS, C_EMB), jnp.float32),
        scratch_shapes=[pltpu.VMEM((HB, C_EMB), jnp.float32)],
    )(ximg_pad, xfused, W1m, b1, W2m, b2)


def _e_body(f_ref, g_ref, idxs_ref, al_ref, w_ref, b_ref, wsT_ref, bs_ref,
            o_ref):
    mask = idxs_ref[...] != DUMP
    x = jnp.where(mask, f_ref[...] + al_ref[...] * g_ref[...], f_ref[...])
    mu = jnp.mean(x, axis=-1, keepdims=True)
    var = jnp.mean((x - mu) ** 2, axis=-1, keepdims=True)
    y = (x - mu) / jnp.sqrt(var + 1e-5) * w_ref[...] + b_ref[...]
    o_ref[...] = jnp.dot(y, wsT_ref[...], preferred_element_type=jnp.float32, precision=lax.Precision.HIGHEST) \
        + bs_ref[...]


def _stage_e(fused, gathered, idx_s, alphac, ln_o_w, ln_o_b, WsT, bs):
    row1 = lambda i: (0, 0)
    return pl.pallas_call(
        _e_body,
        grid=(NBLK,),
        in_specs=[
            pl.BlockSpec((VB, C_EMB), lambda i: (i, 0)),
            pl.BlockSpec((VB, C_EMB), lambda i: (i, 0)),
            pl.BlockSpec((VB, 1), lambda i: (i, 0)),
            pl.BlockSpec((1, 1), row1),
            pl.BlockSpec((1, C_EMB), row1),
            pl.BlockSpec((1, C_EMB), row1),
            pl.BlockSpec((C_EMB, 32), row1),
            pl.BlockSpec((1, 32), row1),
        ],
        out_specs=pl.BlockSpec((VB, 32), lambda i: (i, 0)),
        out_shape=jax.ShapeDtypeStruct((V_PAD, 32), jnp.float32),
    )(fused, gathered, idx_s, alphac, ln_o_w.reshape(1, C_EMB),
      ln_o_b.reshape(1, C_EMB), WsT, bs.reshape(1, 32))


def kernel(point_features, image_features, voxel_coords, calib_matrices,
           img_shape, feat_shape, ln_l_w, ln_l_b, W_l, b_l, ln_i_w, ln_i_b,
           W_i, b_i, Wg1, bg1, Wg2, bg2, Wc1, bc1, Wc2, bc2, alpha,
           ln_o_w, ln_o_b, Ws, bs):
    f32 = jnp.float32
    feat_hw = image_features.transpose(0, 2, 3, 1).reshape(NCAM, HW * HW,
                                                           C_IMG)
    feat_pad = jnp.pad(feat_hw, ((0, 0), (0, KPAD - HW * HW), (0, 0)))
    calib_cat = jnp.transpose(calib_matrices, (2, 0, 1)).reshape(4, 18)
    Wimg = img_shape[1].astype(f32)
    Himg = img_shape[0].astype(f32)
    scal = jnp.stack([2.0 / Wimg, 2.0 / Himg, Wimg, Himg]).reshape(1, 4)
    vox_p = jnp.pad(voxel_coords, ((0, V_PAD - V), (0, 0)))
    pf_p = jnp.pad(point_features, ((0, V_PAD - V), (0, 0)))
    Wg1T = Wg1.T
    fused, img_emb, idx_s, idx_g = _stage_a(
        vox_p, pf_p, feat_pad, calib_cat, scal,
        ln_l_w.reshape(1, 16), ln_l_b.reshape(1, 16), W_l.T,
        b_l.reshape(1, C_EMB),
        ln_i_w.reshape(1, C_IMG), ln_i_b.reshape(1, C_IMG), W_i.T,
        b_i.reshape(1, C_EMB),
        Wg1T[:C_EMB], Wg1T[C_EMB:], bg1.reshape(1, C_EMB),
        Wg2.reshape(1, C_EMB), bg2.reshape(1, 1))

    idx_flat = idx_s.reshape(V_PAD)
    sum_fused = jnp.zeros((TROWS, C_EMB), f32).at[idx_flat].add(fused)
    sum_img = jnp.zeros((TROWS, C_EMB), f32).at[idx_flat].add(img_emb)
    cnt_tbl = jnp.zeros((TROWS, 16), f32).at[idx_flat].add(
        jnp.ones((V_PAD, 16), f32))

    ximg_pad = _c1a(sum_img, cnt_tbl)
    xfused = _c1b(sum_fused, cnt_tbl)
    W1m = jnp.stack([Wc1[:, :, dy + 1, dx + 1].T for dy, dx in SHIFTS])
    W2m = jnp.stack([Wc2[:, :, dy + 1, dx + 1].T for dy, dx in SHIFTS])
    bev_fused = _c2(ximg_pad, xfused, W1m, bc1.reshape(1, C_EMB), W2m,
                    bc2.reshape(1, C_EMB))

    gathered = bev_fused[idx_g.reshape(V_PAD)]

    alphac = jnp.clip(alpha, 0.0, 1.0).reshape(1, 1)
    logits = _stage_e(fused, gathered, idx_s, alphac, ln_o_w, ln_o_b,
                      Ws.T, bs)
    return logits[:V]


# speedup vs baseline: 7.3074x; 1.0099x over previous
import functools

import jax
import jax.numpy as jnp
from jax import lax
from jax.experimental import pallas as pl
from jax.experimental.pallas import tpu as pltpu

V = 100000
VB = 1024
NBLK = 100
V_PAD = VB * NBLK
KPAD = 1408
HW = 37
NCAM = 6
C_IMG = 384
C_EMB = 128
CELLS = 40000
RPAD = 1024
TROWS = 43008
DUMP = 40000
NTB = 40960


def _stage_a(vox_p, pf_p, feat_pad, calib_cat, scal, lnlw, lnlb, WlT, bl,
             lniw, lnib, WiT, bi, Wg1aT, Wg1bT, bg1, wg2row, bg2):
    def body(vox_ref, pf_ref, feat_ref, cc_ref, sc_ref,
             lnlw_ref, lnlb_ref, wlT_ref, bl_ref,
             lniw_ref, lnib_ref, wiT_ref, bi_ref,
             wg1a_ref, wg1b_ref, bg1_ref, wg2_ref, bg2_ref,
             fused_ref, img_ref, idxs_ref, idxg_ref):
        i = pl.program_id(0)
        vox = vox_ref[...]

        def tb(t):
            return t.astype(jnp.bfloat16).astype(jnp.float32)

        vx = tb(vox[:, 0:1]); vy = tb(vox[:, 1:2]); vz = tb(vox[:, 2:3])
        su = sc_ref[0:1, 0:1]; sv = sc_ref[0:1, 1:2]
        wimg = sc_ref[0:1, 2:3]; himg = sc_ref[0:1, 3:4]

        iota_k = lax.broadcasted_iota(jnp.int32, (VB, KPAD), 1)
        acc = jnp.zeros((VB, C_IMG), dtype=jnp.float32)
        cnt = jnp.zeros((VB, 1), dtype=jnp.float32)
        for cam in range(NCAM):
            u = (vx * tb(cc_ref[0:1, 3 * cam:3 * cam + 1])
                 + vy * tb(cc_ref[1:2, 3 * cam:3 * cam + 1])
                 + vz * tb(cc_ref[2:3, 3 * cam:3 * cam + 1])
                 + tb(cc_ref[3:4, 3 * cam:3 * cam + 1]))
            v = (vx * tb(cc_ref[0:1, 3 * cam + 1:3 * cam + 2])
                 + vy * tb(cc_ref[1:2, 3 * cam + 1:3 * cam + 2])
                 + vz * tb(cc_ref[2:3, 3 * cam + 1:3 * cam + 2])
                 + tb(cc_ref[3:4, 3 * cam + 1:3 * cam + 2]))
            w = (vx * tb(cc_ref[0:1, 3 * cam + 2:3 * cam + 3])
                 + vy * tb(cc_ref[1:2, 3 * cam + 2:3 * cam + 3])
                 + vz * tb(cc_ref[2:3, 3 * cam + 2:3 * cam + 3])
                 + tb(cc_ref[3:4, 3 * cam + 2:3 * cam + 3]))
            wc = jnp.maximum(w, 1e-6)
            ud = u / wc
            vd = v / wc
            un = ud * su - 1.0
            vn = vd * sv - 1.0
            x = (un + 1.0) * 0.5 * (HW - 1.0)
            y = (vn + 1.0) * 0.5 * (HW - 1.0)
            x0 = jnp.floor(x); y0 = jnp.floor(y)
            wx1 = x - x0; wx0 = 1.0 - wx1
            wy1 = y - y0; wy0 = 1.0 - wy1
            vx0 = ((x0 >= 0.0) & (x0 <= HW - 1.0)).astype(jnp.float32)
            vx1 = ((x0 + 1.0 >= 0.0) & (x0 + 1.0 <= HW - 1.0)).astype(jnp.float32)
            vy0 = ((y0 >= 0.0) & (y0 <= HW - 1.0)).astype(jnp.float32)
            vy1 = ((y0 + 1.0 >= 0.0) & (y0 + 1.0 <= HW - 1.0)).astype(jnp.float32)
            x0i = jnp.clip(x0, -2.0, 38.0).astype(jnp.int32)
            y0i = jnp.clip(y0, -2.0, 38.0).astype(jnp.int32)
            f00 = y0i * HW + x0i
            w00 = wx0 * wy0 * vx0 * vy0
            w10 = wx1 * wy0 * vx1 * vy0
            w01 = wx0 * wy1 * vx0 * vy1
            w11 = wx1 * wy1 * vx1 * vy1
            s = jnp.where(iota_k == f00, w00, 0.0)
            s = s + jnp.where(iota_k == f00 + 1, w10, 0.0)
            s = s + jnp.where(iota_k == f00 + HW, w01, 0.0)
            s = s + jnp.where(iota_k == f00 + HW + 1, w11, 0.0)
            sampled = jnp.dot(s, feat_ref[cam],
                              preferred_element_type=jnp.float32)
            m = ((w > 0.0) & (ud >= 0.0) & (ud < wimg)
                 & (vd >= 0.0) & (vd < himg)).astype(jnp.float32)
            acc = acc + m * sampled
            cnt = cnt + m
        cntc = jnp.where(cnt == 0.0, 1.0, cnt)
        vis = acc / cntc

        mu = jnp.mean(vis, axis=-1, keepdims=True)
        var = jnp.mean((vis - mu) ** 2, axis=-1, keepdims=True)
        lnv = (vis - mu) / jnp.sqrt(var + 1e-5) * lniw_ref[...] + lnib_ref[...]
        img = jax.nn.relu(
            jnp.dot(lnv, wiT_ref[...], preferred_element_type=jnp.float32, precision=lax.Precision.HIGHEST)
            + bi_ref[...])

        pf = pf_ref[...]
        mul = jnp.mean(pf, axis=-1, keepdims=True)
        varl = jnp.mean((pf - mul) ** 2, axis=-1, keepdims=True)
        lnp = (pf - mul) / jnp.sqrt(varl + 1e-5) * lnlw_ref[...] + lnlb_ref[...]
        lid = jax.nn.relu(
            jnp.dot(lnp, wlT_ref[...], preferred_element_type=jnp.float32, precision=lax.Precision.HIGHEST)
            + bl_ref[...])

        g_h = jax.nn.relu(
            jnp.dot(lid, wg1a_ref[...], preferred_element_type=jnp.float32, precision=lax.Precision.HIGHEST)
            + jnp.dot(img, wg1b_ref[...], preferred_element_type=jnp.float32, precision=lax.Precision.HIGHEST)
            + bg1_ref[...])
        gate = jax.nn.sigmoid(
            jnp.sum(g_h * wg2_ref[...], axis=-1, keepdims=True) + bg2_ref[...])
        fused = gate * img + (1.0 - gate) * lid

        bx = vox[:, 0:1]
        by = vox[:, 1:2]
        ix = jnp.floor((bx + 50.0) * 2.0).astype(jnp.int32)
        iy = jnp.floor((by + 50.0) * 2.0).astype(jnp.int32)
        inb = (ix >= 0) & (ix < 200) & (iy >= 0) & (iy < 200)
        rowid = i * VB + lax.broadcasted_iota(jnp.int32, (VB, 1), 0)
        realrow = rowid < V
        idxf = iy * 200 + ix
        idxs_ref[...] = jnp.where(inb & realrow, idxf, DUMP)
        idxg_ref[...] = jnp.where(inb & realrow, idxf, 0)
        fused_ref[...] = fused
        img_ref[...] = img

    row1 = lambda i: (0, 0)
    return pl.pallas_call(
        body,
        grid=(NBLK,),
        in_specs=[
            pl.BlockSpec((VB, 3), lambda i: (i, 0)),
            pl.BlockSpec((VB, 16), lambda i: (i, 0)),
            pl.BlockSpec((NCAM, KPAD, C_IMG), lambda i: (0, 0, 0)),
            pl.BlockSpec((4, 18), row1),
            pl.BlockSpec((1, 4), row1),
            pl.BlockSpec((1, 16), row1),
            pl.BlockSpec((1, 16), row1),
            pl.BlockSpec((16, C_EMB), row1),
            pl.BlockSpec((1, C_EMB), row1),
            pl.BlockSpec((1, C_IMG), row1),
            pl.BlockSpec((1, C_IMG), row1),
            pl.BlockSpec((C_IMG, C_EMB), row1),
            pl.BlockSpec((1, C_EMB), row1),
            pl.BlockSpec((C_EMB, C_EMB), row1),
            pl.BlockSpec((C_EMB, C_EMB), row1),
            pl.BlockSpec((1, C_EMB), row1),
            pl.BlockSpec((1, C_EMB), row1),
            pl.BlockSpec((1, 1), row1),
        ],
        out_specs=[
            pl.BlockSpec((VB, C_EMB), lambda i: (i, 0)),
            pl.BlockSpec((VB, C_EMB), lambda i: (i, 0)),
            pl.BlockSpec((VB, 1), lambda i: (i, 0)),
            pl.BlockSpec((VB, 1), lambda i: (i, 0)),
        ],
        out_shape=[
            jax.ShapeDtypeStruct((V_PAD, C_EMB), jnp.float32),
            jax.ShapeDtypeStruct((V_PAD, C_EMB), jnp.float32),
            jax.ShapeDtypeStruct((V_PAD, 1), jnp.int32),
            jax.ShapeDtypeStruct((V_PAD, 1), jnp.int32),
        ],
    )(vox_p, pf_p, feat_pad, calib_cat, scal, lnlw, lnlb, WlT, bl,
      lniw, lnib, WiT, bi, Wg1aT, Wg1bT, bg1, wg2row, bg2)


from jax.experimental.pallas import tpu_sc as plsc

NSUB = 16
NCORE = 2
ROWS_PER_SUB = NTB // NSUB
PTS_PER_SUB = V_PAD // NSUB
SBLK = 128
BLKS_PER_SUB = PTS_PER_SUB // SBLK
NW = NSUB * NCORE
GPTS_PER_W = V_PAD // NW
GBLKS_PER_W = GPTS_PER_W // SBLK


def _sc_gather(tbl, idx32):
    mesh = plsc.VectorSubcoreMesh(core_axis_name="c", subcore_axis_name="s")

    @functools.partial(
        pl.kernel, mesh=mesh,
        out_type=jax.ShapeDtypeStruct((V_PAD, C_EMB), jnp.float32),
        scratch_types=[
            pltpu.VMEM((GBLKS_PER_W, SBLK), jnp.int32),
            pltpu.VMEM((SBLK, C_EMB), jnp.float32),
            pltpu.SemaphoreType.DMA,
        ],
    )
    def k(tbl_hbm, idx_hbm, out_hbm, idx_v, rbuf, sem):
        core = lax.axis_index("c")
        sid = lax.axis_index("s")
        wid = sid * NCORE + core
        pltpu.sync_copy(idx_hbm.at[wid], idx_v)

        def body(j, carry):
            pltpu.async_copy(tbl_hbm.at[idx_v.at[j]], rbuf, sem).wait()
            pltpu.sync_copy(
                rbuf, out_hbm.at[pl.ds(wid * GPTS_PER_W + j * SBLK, SBLK), :])
            return carry
        lax.fori_loop(0, GBLKS_PER_W, body, 0)

    return k(tbl, idx32)


def _c1a_body(simg_ref, cnt_ref, out_ref):
    i = pl.program_id(0)
    p = i * 1024 + lax.broadcasted_iota(jnp.int32, (1024, 1), 0)
    valid = (p >= RPAD) & (p < RPAD + CELLS)
    c = cnt_ref[:, 0:1]
    cc = jnp.where(c == 0.0, 1.0, c)
    out_ref[...] = jnp.where(valid, simg_ref[...] / cc, 0.0)


def _c1a(sum_img, cnt_tbl):
    return pl.pallas_call(
        _c1a_body,
        grid=(TROWS // 1024,),
        in_specs=[
            pl.BlockSpec((1024, C_EMB), lambda i: (i, 0)),
            pl.BlockSpec((1024, 16), lambda i: (i, 0)),
        ],
        out_specs=pl.BlockSpec((1024, C_EMB), lambda i: (i, 0)),
        out_shape=jax.ShapeDtypeStruct((TROWS, C_EMB), jnp.float32),
    )(sum_img, cnt_tbl)


def _c1b_body(sf_ref, cnt_ref, out_ref):
    c = cnt_ref[:, 0:1]
    cc = jnp.where(c == 0.0, 1.0, c)
    out_ref[...] = sf_ref[...] / cc


def _c1b(sum_fused, cnt_tbl):
    return pl.pallas_call(
        _c1b_body,
        grid=(40960 // 1024,),
        in_specs=[
            pl.BlockSpec((1024, C_EMB), lambda i: (i + 1, 0)),
            pl.BlockSpec((1024, 16), lambda i: (i + 1, 0)),
        ],
        out_specs=pl.BlockSpec((1024, C_EMB), lambda i: (i, 0)),
        out_shape=jax.ShapeDtypeStruct((40960, C_EMB), jnp.float32),
    )(sum_fused, cnt_tbl)


RB = 4000
HB = 4408
SHIFTS = [(dy, dx) for dy in (-1, 0, 1) for dx in (-1, 0, 1)]


def _c2_body(ximg_ref, xf_ref, w1_ref, b1_ref, w2_ref, b2_ref, out_ref,
             h_ref):
    i = pl.program_id(0)
    r0 = i * RB
    jj = lax.broadcasted_iota(jnp.int32, (HB, 1), 0)
    q2 = (r0 - 204) + jj + 1600
    wq = lax.rem(q2, 200)
    acc = jnp.zeros((HB, C_EMB), dtype=jnp.float32)
    for k, (dy, dx) in enumerate(SHIFTS):
        sft = dy * 200 + dx
        start = RPAD + (r0 - 204) + sft
        xs = ximg_ref[pl.ds(start, HB), :]
        t = jnp.dot(xs, w1_ref[k], preferred_element_type=jnp.float32)
        if dx == 1:
            t = jnp.where(wq == 199, 0.0, t)
        elif dx == -1:
            t = jnp.where(wq == 0, 0.0, t)
        acc = acc + t
    h_ref[...] = jax.nn.relu(acc + b1_ref[...])

    kk = lax.broadcasted_iota(jnp.int32, (RB, 1), 0)
    p = r0 + kk
    wp = lax.rem(p, 200)
    acc2 = jnp.zeros((RB, C_EMB), dtype=jnp.float32)
    for k, (dy, dx) in enumerate(SHIFTS):
        sft = dy * 200 + dx
        hs = h_ref[pl.ds(204 + sft, RB), :]
        t = jnp.dot(hs, w2_ref[k], preferred_element_type=jnp.float32)
        okr = None
        if dy == -1:
            okr = p >= 200
        elif dy == 1:
            okr = p < CELLS - 200
        okc = None
        if dx == 1:
            okc = wp != 199
        elif dx == -1:
            okc = wp != 0
        if okr is not None and okc is not None:
            t = jnp.where(okr & okc, t, 0.0)
        elif okr is not None:
            t = jnp.where(okr, t, 0.0)
        elif okc is not None:
            t = jnp.where(okc, t, 0.0)
        acc2 = acc2 + t
    out_ref[...] = xf_ref[...] + 0.5 * (acc2 + b2_ref[...])


def _c2(ximg_pad, xfused, W1m, b1, W2m, b2):
    return pl.pallas_call(
        _c2_body,
        grid=(CELLS // RB,),
        in_specs=[
            pl.BlockSpec((TROWS, C_EMB), lambda i: (0, 0)),
            pl.BlockSpec((RB, C_EMB), lambda i: (i, 0)),
            pl.BlockSpec((9, C_EMB, C_EMB), lambda i: (0, 0, 0)),
            pl.BlockSpec((1, C_EMB), lambda i: (0, 0)),
            pl.BlockSpec((9, C_EMB, C_EMB), lambda i: (0, 0, 0)),
            pl.BlockSpec((1, C_EMB), lambda i: (0, 0)),
        ],
        out_specs=pl.BlockSpec((RB, C_EMB), lambda i: (i, 0)),
        out_shape=jax.ShapeDtypeStruct((CELLS, C_EMB), jnp.float32),
        scratch_shapes=[pltpu.VMEM((HB, C_EMB), jnp.float32)],
    )(ximg_pad, xfused, W1m, b1, W2m, b2)


def _e_body(f_ref, g_ref, idxs_ref, al_ref, w_ref, b_ref, wsT_ref, bs_ref,
            o_ref):
    mask = idxs_ref[...] != DUMP
    fused = f_ref[...]
    x = jnp.where(mask, fused + al_ref[...] * g_ref[...], fused)
    mu = jnp.mean(x, axis=-1, keepdims=True)
    var = jnp.mean((x - mu) ** 2, axis=-1, keepdims=True)
    y = (x - mu) / jnp.sqrt(var + 1e-5) * w_ref[...] + b_ref[...]
    o_ref[...] = jnp.dot(y, wsT_ref[...], preferred_element_type=jnp.float32, precision=lax.Precision.HIGHEST) \
        + bs_ref[...]


def _stage_e(fused, gathered, idx_s, alphac, ln_o_w, ln_o_b, WsT, bs):
    row1 = lambda i: (0, 0)
    return pl.pallas_call(
        _e_body,
        grid=(NBLK,),
        in_specs=[
            pl.BlockSpec((VB, C_EMB), lambda i: (i, 0)),
            pl.BlockSpec((VB, C_EMB), lambda i: (i, 0)),
            pl.BlockSpec((VB, 1), lambda i: (i, 0)),
            pl.BlockSpec((1, 1), row1),
            pl.BlockSpec((1, C_EMB), row1),
            pl.BlockSpec((1, C_EMB), row1),
            pl.BlockSpec((C_EMB, 32), row1),
            pl.BlockSpec((1, 32), row1),
        ],
        out_specs=pl.BlockSpec((VB, 32), lambda i: (i, 0)),
        out_shape=jax.ShapeDtypeStruct((V_PAD, 32), jnp.float32),
    )(fused, gathered, idx_s, alphac, ln_o_w.reshape(1, C_EMB),
      ln_o_b.reshape(1, C_EMB), WsT, bs.reshape(1, 32))


def kernel(point_features, image_features, voxel_coords, calib_matrices,
           img_shape, feat_shape, ln_l_w, ln_l_b, W_l, b_l, ln_i_w, ln_i_b,
           W_i, b_i, Wg1, bg1, Wg2, bg2, Wc1, bc1, Wc2, bc2, alpha,
           ln_o_w, ln_o_b, Ws, bs):
    f32 = jnp.float32
    feat_hw = image_features.transpose(0, 2, 3, 1).reshape(NCAM, HW * HW,
                                                           C_IMG)
    feat_pad = jnp.pad(feat_hw, ((0, 0), (0, KPAD - HW * HW), (0, 0)))
    calib_cat = jnp.transpose(calib_matrices, (2, 0, 1)).reshape(4, 18)
    Wimg = img_shape[1].astype(f32)
    Himg = img_shape[0].astype(f32)
    scal = jnp.stack([2.0 / Wimg, 2.0 / Himg, Wimg, Himg]).reshape(1, 4)
    vox_p = jnp.pad(voxel_coords, ((0, V_PAD - V), (0, 0)))
    pf_p = jnp.pad(point_features, ((0, V_PAD - V), (0, 0)))
    Wg1T = Wg1.T
    fused_cs, img_cs, idx_s, idx_g = _stage_a(
        vox_p, pf_p, feat_pad, calib_cat, scal,
        ln_l_w.reshape(1, 16), ln_l_b.reshape(1, 16), W_l.T,
        b_l.reshape(1, C_EMB),
        ln_i_w.reshape(1, C_IMG), ln_i_b.reshape(1, C_IMG), W_i.T,
        b_i.reshape(1, C_EMB),
        Wg1T[:C_EMB], Wg1T[C_EMB:], bg1.reshape(1, C_EMB),
        Wg2.reshape(1, C_EMB), bg2.reshape(1, 1))

    idx_flat = idx_s.reshape(V_PAD) + RPAD
    sum_fused = jnp.zeros((TROWS, C_EMB), f32).at[idx_flat].add(fused_cs)
    sum_img = jnp.zeros((TROWS, C_EMB), f32).at[idx_flat].add(img_cs)
    cnt_tbl = jnp.zeros((TROWS, 16), f32).at[idx_flat].add(
        jnp.ones((V_PAD, 16), f32))

    ximg_pad = _c1a(sum_img, cnt_tbl)
    xfused = _c1b(sum_fused, cnt_tbl)
    W1m = jnp.stack([Wc1[:, :, dy + 1, dx + 1].T for dy, dx in SHIFTS])
    W2m = jnp.stack([Wc2[:, :, dy + 1, dx + 1].T for dy, dx in SHIFTS])
    bev_fused = _c2(ximg_pad, xfused, W1m, bc1.reshape(1, C_EMB), W2m,
                    bc2.reshape(1, C_EMB))

    gathered = _sc_gather(bev_fused, idx_g.reshape(NW, GBLKS_PER_W, SBLK))

    alphac = jnp.clip(alpha, 0.0, 1.0).reshape(1, 1)
    logits = _stage_e(fused_cs, gathered, idx_s, alphac, ln_o_w, ln_o_b,
                      Ws.T, bs)
    return logits[:V]
